# baseline (speedup 1.0000x reference)
import jax
import jax.numpy as jnp
from jax.experimental import pallas as pl
from jax.experimental.pallas import tpu as pltpu
from jax.experimental.pallas import tpu_sc as plsc

EMBED_DIM = 64
HIST = 50
SCALE = 8.0
LANES = 16

NC, NS = 2, 16
NW = NC * NS
IPC = 2 * HIST
IROW = 128
NGRP = 7
XPC = 2
NBUF = 2


def _sc_gather_scale(pair_table, idx, batch):
    num_rows = idx.shape[0]
    cpw = num_rows // NW

    mesh = plsc.VectorSubcoreMesh(core_axis_name="c", subcore_axis_name="s")

    @pl.kernel(
        out_type=jax.ShapeDtypeStruct((batch, HIST, EMBED_DIM), jnp.float32),
        mesh=mesh,
        scratch_types=[
            pltpu.VMEM((cpw, IROW), jnp.int32),
            pltpu.VMEM((NBUF, NGRP * LANES), jnp.int32),
            pltpu.VMEM((NBUF, NGRP * LANES, 2 * EMBED_DIM), jnp.float32),
            pltpu.VMEM((NBUF, IPC, EMBED_DIM), jnp.float32),
            pltpu.SemaphoreType.DMA((NBUF,)),
            pltpu.SemaphoreType.DMA((NBUF,)),
        ],
        compiler_params=pltpu.CompilerParams(
            use_tc_tiling_on_sc=True, needs_layout_passes=False
        ),
    )
    def k(tab_hbm, i_hbm, o_hbm, raw_v, hidx_v, in_v, out_v, gsem, wsem):
        wid = jax.lax.axis_index("s") * NC + jax.lax.axis_index("c")

        pltpu.sync_copy(i_hbm.at[pl.ds(wid * cpw, cpw)], raw_v)

        iota = jax.lax.iota(jnp.int32, LANES)
        zero = iota * 0
        posl = [jnp.minimum(iota + g * LANES, IPC - 1) for g in range(NGRP)]
        tailmask = iota < (IPC - (NGRP - 1) * LANES)

        def start_gather(cc, b):
            for g in range(NGRP):
                hidx_v.at[b, pl.ds(g * LANES, LANES)][...] = (
                    raw_v.at[cc, pl.ds(g * LANES, LANES)][...] >> 1
                )
            pltpu.async_copy(
                tab_hbm.at[hidx_v.at[b, pl.ds(0, IPC)]],
                in_v.at[b, pl.ds(0, IPC)],
                gsem.at[b],
            )

        def wait_gather(b):
            pltpu.make_async_copy(
                tab_hbm.at[hidx_v.at[b, pl.ds(0, IPC)]],
                in_v.at[b, pl.ds(0, IPC)],
                gsem.at[b],
            ).wait()

        def scale_half(cc, b):
            for g in range(NGRP):
                rvec = raw_v[cc, pl.ds(g * LANES, LANES)]
                offv = (rvec & 1) << 6
                mask = tailmask if g == NGRP - 1 else None
                for c in range(EMBED_DIM):
                    vals = plsc.load_gather(in_v.at[b], [posl[g], offv + c])
                    plsc.store_scatter(
                        out_v.at[b], [posl[g], zero + c], vals * SCALE,
                        mask=mask,
                    )

        def write_out(cc, b):
            xr = (wid * cpw + cc) * XPC
            for s in range(XPC):
                pltpu.async_copy(
                    out_v.at[b, pl.ds(s * HIST, HIST)],
                    o_hbm.at[xr + s],
                    wsem.at[b],
                )

        def wait_write(cc, b):
            xr = (wid * cpw + cc) * XPC
            for s in range(XPC):
                pltpu.make_async_copy(
                    out_v.at[b, pl.ds(s * HIST, HIST)],
                    o_hbm.at[xr + s],
                    wsem.at[b],
                ).wait()

        for b in range(NBUF):
            start_gather(b, b)
        for b in range(NBUF):
            wait_gather(b)
            scale_half(b, b)
            write_out(b, b)
            start_gather(NBUF + b, b)

        @pl.loop(1, cpw // NBUF)
        def _(r):
            for b in range(NBUF):
                cc = r * NBUF + b
                wait_gather(b)
                wait_write(cc - NBUF, b)
                scale_half(cc, b)
                write_out(cc, b)

                @pl.when(cc + NBUF < cpw)
                def _():
                    start_gather(cc + NBUF, b)

        for b in range(NBUF):
            wait_write(cpw - NBUF + b, b)

    return k(pair_table, idx)


def kernel(x, table):
    b, h = x.shape
    v, d = table.shape
    idx = x.astype(jnp.int32).reshape(b * h // IPC, IPC)
    idx = jnp.pad(idx, ((0, 0), (0, IROW - IPC)))
    pair_table = table.reshape(v // 2, 2 * d)
    return _sc_gather_scale(pair_table, idx, b)


# speedup vs baseline: 2.3473x; 2.3473x over previous
import jax
import jax.numpy as jnp
from jax.experimental import pallas as pl
from jax.experimental.pallas import tpu as pltpu
from jax.experimental.pallas import tpu_sc as plsc

EMBED_DIM = 64
HIST = 50
SCALE = 8.0
LANES = 16

NC, NS = 2, 16
NW = NC * NS
IPC = 2 * HIST
XPC = 2
NBUF = 2


def _sc_gather_scale(padded_table, idx, batch):
    num_rows = idx.shape[0]
    cpw = num_rows // NW
    pd = padded_table.shape[1]

    mesh = plsc.VectorSubcoreMesh(core_axis_name="c", subcore_axis_name="s")

    @pl.kernel(
        out_type=jax.ShapeDtypeStruct((batch, HIST, EMBED_DIM), jnp.float32),
        mesh=mesh,
        scratch_types=[
            pltpu.VMEM((cpw, IPC), jnp.int32),
            pltpu.VMEM((NBUF, IPC, pd), jnp.float32),
            pltpu.VMEM((NBUF, XPC, HIST, EMBED_DIM), jnp.float32),
            pltpu.SemaphoreType.DMA((NBUF,)),
            pltpu.SemaphoreType.DMA((NBUF,)),
        ],
        compiler_params=pltpu.CompilerParams(use_tc_tiling_on_sc=True),
    )
    def k(tab_hbm, i_hbm, o_hbm, idx_v, in_v, out_v, gsem, wsem):
        wid = jax.lax.axis_index("s") * NC + jax.lax.axis_index("c")

        pltpu.sync_copy(i_hbm.at[pl.ds(wid * cpw, cpw)], idx_v)

        def start_gather(cc, b):
            pltpu.async_copy(
                tab_hbm.at[idx_v.at[cc]], in_v.at[b], gsem.at[b]
            )

        def wait_gather(cc, b):
            pltpu.make_async_copy(
                tab_hbm.at[idx_v.at[cc]], in_v.at[b], gsem.at[b]
            ).wait()

        def scale(b):
            for s in range(XPC):
                @pl.loop(0, HIST)
                def _(rr):
                    for c in range(0, EMBED_DIM, LANES):
                        out_v.at[b, s, rr, pl.ds(c, LANES)][...] = (
                            in_v.at[b, s * HIST + rr, pl.ds(c, LANES)][...]
                            * SCALE
                        )

        def write_dst(cc):
            return o_hbm.at[pl.ds((wid * cpw + cc) * XPC, XPC)]

        for b in range(NBUF):
            start_gather(b, b)
        for b in range(NBUF):
            wait_gather(b, b)
            scale(b)
            pltpu.async_copy(out_v.at[b], write_dst(b), wsem.at[b])
            start_gather(NBUF + b, b)

        @pl.loop(1, cpw // NBUF)
        def _(r):
            for b in range(NBUF):
                cc = r * NBUF + b
                wait_gather(cc, b)
                pltpu.make_async_copy(
                    out_v.at[b], write_dst(cc - NBUF), wsem.at[b]
                ).wait()
                scale(b)
                pltpu.async_copy(out_v.at[b], write_dst(cc), wsem.at[b])

                @pl.when(cc + NBUF < cpw)
                def _():
                    start_gather(cc + NBUF, b)

        for b in range(NBUF):
            pltpu.make_async_copy(
                out_v.at[b], write_dst(cpw - NBUF + b), wsem.at[b]
            ).wait()

    return k(padded_table, idx)


def kernel(x, table):
    b, h = x.shape
    v, d = table.shape
    idx = x.astype(jnp.int32).reshape(b * h // IPC, IPC)
    padded_table = jnp.concatenate([table, table], axis=1)
    return _sc_gather_scale(padded_table, idx, b)
